# baseline (speedup 1.0000x reference)
import jax
import jax.numpy as jnp
from jax.experimental import pallas as pl

_TILE_P = 512


def _mlp_kernel(x_ref, wa_ref, wb_ref, b0_ref, w1_ref, b1_ref, out_ref):
    h = jnp.dot(x_ref[0], wa_ref[...], preferred_element_type=jnp.float32)
    h += jnp.dot(x_ref[1], wb_ref[...], preferred_element_type=jnp.float32)
    h = jnp.maximum(h + b0_ref[...], 0.0)
    out = jnp.dot(h, w1_ref[...], preferred_element_type=jnp.float32)
    out_ref[...] = out + b1_ref[...]


def kernel(inputs, W0, b0, W1, b1):
    _, batch, k = inputs.shape
    h1 = W0.shape[1]
    p = 128 // k
    bp = batch // p
    xp = inputs.reshape(2, bp, p * k)

    eye = jnp.eye(p, dtype=W0.dtype)
    wa = jnp.einsum("pq,kh->pkqh", eye, W0[:k]).reshape(p * k, p * h1)
    wb = jnp.einsum("pq,kh->pkqh", eye, W0[k:]).reshape(p * k, p * h1)
    b0r = jnp.tile(b0, p).reshape(1, p * h1)
    w1e = jnp.einsum("pq,h->phq", eye, W1[:, 0]).reshape(p * h1, p)
    b1r = b1.reshape(1, 1)

    tile = min(_TILE_P, bp)
    grid = (bp // tile,)
    out_p = pl.pallas_call(
        _mlp_kernel,
        grid=grid,
        in_specs=[
            pl.BlockSpec((2, tile, p * k), lambda i: (0, i, 0)),
            pl.BlockSpec((p * k, p * h1), lambda i: (0, 0)),
            pl.BlockSpec((p * k, p * h1), lambda i: (0, 0)),
            pl.BlockSpec((1, p * h1), lambda i: (0, 0)),
            pl.BlockSpec((p * h1, p), lambda i: (0, 0)),
            pl.BlockSpec((1, 1), lambda i: (0, 0)),
        ],
        out_specs=pl.BlockSpec((tile, p), lambda i: (i, 0)),
        out_shape=jax.ShapeDtypeStruct((bp, p), jnp.float32),
    )(xp, wa, wb, b0r, w1e, b1r)
    return out_p.reshape(batch, 1)


# speedup vs baseline: 1.2906x; 1.2906x over previous
import jax
import jax.numpy as jnp
from jax.experimental import pallas as pl


def _mlp_kernel(x_ref, w0a_ref, w0b_ref, b0_ref, w1_ref, b1_ref, out_ref):
    h = jnp.dot(x_ref[0], w0a_ref[...], preferred_element_type=jnp.float32)
    h += jnp.dot(x_ref[1], w0b_ref[...], preferred_element_type=jnp.float32)
    h = jnp.maximum(h + b0_ref[...], 0.0)
    out_ref[...] = jnp.sum(h * w1_ref[...], axis=1, keepdims=True) + b1_ref[...]


def kernel(inputs, W0, b0, W1, b1):
    _, batch, k = inputs.shape
    h1 = W0.shape[1]
    w0a = W0[:k]
    w0b = W0[k:]
    b0r = b0.reshape(1, h1)
    w1r = W1.reshape(1, h1)
    b1r = b1.reshape(1, 1)
    return pl.pallas_call(
        _mlp_kernel,
        out_shape=jax.ShapeDtypeStruct((batch, 1), jnp.float32),
    )(inputs, w0a, w0b, b0r, w1r, b1r)
